# Initial kernel scaffold; baseline (speedup 1.0000x reference)
import jax
import jax.numpy as jnp
from jax.experimental import pallas as pl


def kernel(x, edge_index, edge_types, graph_ids, W_e, b_e, W_ih, W_hh, b_ih, b_hh, Wc, bc):
    raise NotImplementedError("write your pallas kernel here")


# speedup vs baseline: 16.2178x; 16.2178x over previous
import functools

import jax
import jax.numpy as jnp
from jax import lax
from jax.experimental import pallas as pl
from jax.experimental.pallas import tpu as pltpu
from jax.experimental.pallas import tpu_sc as plsc

STEPS = 8
G = 128

N = 10000
E = 320000
D = 128
T = 3

BN = 1000
NB = N // BN

NSC = 2
NSUB = 16
NW = NSC * NSUB
K = 128
NCH = 79
EPT = K * NCH
EP = EPT * NW
RSUB = 632
NACC = NSUB * RSUB


def _gru_math(a, h, wih_ref, whh_ref, bih_ref, bhh_ref):
    gi = lax.dot_general(a, wih_ref[...], (((1,), (1,)), ((), ())),
                         preferred_element_type=jnp.float32) + bih_ref[...][None, :]
    gh = lax.dot_general(h, whh_ref[...], (((1,), (1,)), ((), ())),
                         preferred_element_type=jnp.float32) + bhh_ref[...][None, :]
    r = jax.nn.sigmoid(gi[:, :D] + gh[:, :D])
    z = jax.nn.sigmoid(gi[:, D:2 * D] + gh[:, D:2 * D])
    n = jnp.tanh(gi[:, 2 * D:] + r * gh[:, 2 * D:])
    return (1.0 - z) * n + z * h


def _proj_body(h_ref, we_ref, be_ref, p_ref):
    h = h_ref[...]
    p = lax.dot_general(h, we_ref[0], (((1,), (1,)), ((), ())),
                        preferred_element_type=jnp.float32)
    p_ref[0] = p + be_ref[0, 0][None, :]


def _proj_call(h, W_e, be3):
    return pl.pallas_call(
        _proj_body,
        grid=(T, NB),
        in_specs=[
            pl.BlockSpec((BN, D), lambda t, i: (i, 0)),
            pl.BlockSpec((1, D, D), lambda t, i: (t, 0, 0)),
            pl.BlockSpec((1, 1, D), lambda t, i: (t, 0, 0)),
        ],
        out_specs=pl.BlockSpec((1, BN, D), lambda t, i: (t, i, 0)),
        out_shape=jax.ShapeDtypeStruct((T, N, D), jnp.float32),
    )(h, W_e, be3)


def _sc_agg_body(p_hbm, idx_hbm, dst_hbm, zeros_hbm, out_hbm,
                 idx_v, dst_v, rows_v, acc_sh, sem):
    c = lax.axis_index("c")
    s = lax.axis_index("s")
    w = c * NSUB + s
    pltpu.sync_copy(zeros_hbm.at[pl.ds(s * RSUB, RSUB)],
                    acc_sh.at[pl.ds(s * RSUB, RSUB)])
    plsc.subcore_barrier()
    base = w * EPT

    def body(i, carry):
        off = pl.multiple_of(base + i * K, 8)
        pltpu.sync_copy(idx_hbm.at[pl.ds(off, K)], idx_v)
        pltpu.async_copy(p_hbm.at[idx_v], rows_v, sem).wait()
        pltpu.sync_copy(dst_hbm.at[pl.ds(off, K)], dst_v)
        pltpu.sync_copy(rows_v, acc_sh.at[dst_v], add=True)
        return carry

    lax.fori_loop(0, NCH, body, 0)
    plsc.subcore_barrier()
    pltpu.sync_copy(acc_sh.at[pl.ds(s * RSUB, RSUB)],
                    out_hbm.at[c].at[pl.ds(s * RSUB, RSUB)])


_sc_agg = functools.partial(
    pl.kernel,
    mesh=plsc.VectorSubcoreMesh(core_axis_name="c", subcore_axis_name="s",
                                num_cores=NSC, num_subcores=NSUB),
    out_type=jax.ShapeDtypeStruct((NSC, NACC, D), jnp.float32),
    scratch_types=[
        pltpu.VMEM((K,), jnp.int32),
        pltpu.VMEM((K,), jnp.int32),
        pltpu.VMEM((K, D), jnp.float32),
        pltpu.VMEM_SHARED((NACC, D), jnp.float32),
        pltpu.SemaphoreType.DMA,
    ],
)(_sc_agg_body)


def _gru_proj_body(parts_ref, h_ref, wih_ref, whh_ref, bih_ref, bhh_ref,
                   we_ref, be_ref, hnew_ref, p_ref):
    a = parts_ref[0] + parts_ref[1]
    h = h_ref[...]
    h_new = _gru_math(a, h, wih_ref, whh_ref, bih_ref, bhh_ref)
    hnew_ref[...] = h_new
    for t in range(T):
        p = lax.dot_general(h_new, we_ref[t], (((1,), (1,)), ((), ())),
                            preferred_element_type=jnp.float32)
        p_ref[t] = p + be_ref[t, 0][None, :]


def _gru_proj_call(parts, h, W_ih, W_hh, b_ih, b_hh, W_e, be3):
    return pl.pallas_call(
        _gru_proj_body,
        grid=(NB,),
        in_specs=[
            pl.BlockSpec((NSC, BN, D), lambda i: (0, i, 0)),
            pl.BlockSpec((BN, D), lambda i: (i, 0)),
            pl.BlockSpec((3 * D, D), lambda i: (0, 0)),
            pl.BlockSpec((3 * D, D), lambda i: (0, 0)),
            pl.BlockSpec((3 * D,), lambda i: (0,)),
            pl.BlockSpec((3 * D,), lambda i: (0,)),
            pl.BlockSpec((T, D, D), lambda i: (0, 0, 0)),
            pl.BlockSpec((T, 1, D), lambda i: (0, 0, 0)),
        ],
        out_specs=[
            pl.BlockSpec((BN, D), lambda i: (i, 0)),
            pl.BlockSpec((T, BN, D), lambda i: (0, i, 0)),
        ],
        out_shape=[
            jax.ShapeDtypeStruct((N, D), jnp.float32),
            jax.ShapeDtypeStruct((T, N, D), jnp.float32),
        ],
    )(parts, h, W_ih, W_hh, b_ih, b_hh, W_e, be3)


def _gru_pool_body(parts_ref, h_ref, wih_ref, whh_ref, bih_ref, bhh_ref,
                   gids_ref, wc_ref, bc_ref, res_ref, ggnn_ref, acc_ref):
    i = pl.program_id(0)
    a = parts_ref[0] + parts_ref[1]
    h = h_ref[...]
    h_new = _gru_math(a, h, wih_ref, whh_ref, bih_ref, bhh_ref)
    gids = gids_ref[0, 0, :]
    onehot = (lax.broadcasted_iota(jnp.int32, (G, BN), 0)
              == gids[None, :]).astype(jnp.float32)
    contrib = lax.dot_general(onehot, h_new, (((1,), (0,)), ((), ())),
                              preferred_element_type=jnp.float32, precision=lax.Precision.HIGHEST)

    @pl.when(i == 0)
    def _():
        acc_ref[...] = contrib

    @pl.when(i > 0)
    def _():
        acc_ref[...] = acc_ref[...] + contrib

    @pl.when(i == NB - 1)
    def _():
        acc = acc_ref[...]
        wc = wc_ref[...]
        gg_col = lax.dot_general(acc, wc, (((1,), (1,)), ((), ())),
                                 preferred_element_type=jnp.float32)
        gg_row = lax.dot_general(wc, acc, (((1,), (1,)), ((), ())),
                                 preferred_element_type=jnp.float32)
        ggnn_ref[...] = gg_col[:, 0:1] + bc_ref[0]
        res_ref[...] = jax.nn.sigmoid(gg_row + bc_ref[0])[0]


def _gru_pool_call(parts, h, W_ih, W_hh, b_ih, b_hh, gids3, wcv, bc):
    return pl.pallas_call(
        _gru_pool_body,
        grid=(NB,),
        in_specs=[
            pl.BlockSpec((NSC, BN, D), lambda i: (0, i, 0)),
            pl.BlockSpec((BN, D), lambda i: (i, 0)),
            pl.BlockSpec((3 * D, D), lambda i: (0, 0)),
            pl.BlockSpec((3 * D, D), lambda i: (0, 0)),
            pl.BlockSpec((3 * D,), lambda i: (0,)),
            pl.BlockSpec((3 * D,), lambda i: (0,)),
            pl.BlockSpec((1, 1, BN), lambda i: (i, 0, 0)),
            pl.BlockSpec((8, D), lambda i: (0, 0)),
            pl.BlockSpec(memory_space=pltpu.SMEM),
        ],
        out_specs=[
            pl.BlockSpec((G,), lambda i: (0,)),
            pl.BlockSpec((G, 1), lambda i: (0, 0)),
        ],
        out_shape=[
            jax.ShapeDtypeStruct((G,), jnp.float32),
            jax.ShapeDtypeStruct((G, 1), jnp.float32),
        ],
        scratch_shapes=[pltpu.VMEM((G, D), jnp.float32)],
    )(parts, h, W_ih, W_hh, b_ih, b_hh, gids3, wcv, bc)


def kernel(x, edge_index, edge_types, graph_ids, W_e, b_e, W_ih, W_hh,
           b_ih, b_hh, Wc, bc):
    src = edge_index[0]
    dst = edge_index[1]
    pad = EP - E
    flat_idx = edge_types * N + src
    idx_p = jnp.concatenate([flat_idx, jnp.zeros((pad,), jnp.int32)])
    dst_p = jnp.concatenate([dst, jnp.full((pad,), N, jnp.int32)])
    zeros_acc = jnp.zeros((NACC, D), jnp.float32)
    gids3 = graph_ids.reshape(NB, 1, BN)
    be3 = b_e.reshape(T, 1, D)
    wcv = jnp.pad(Wc, ((0, 7), (0, 0)))

    h = x
    P = _proj_call(h, W_e, be3)
    for step in range(STEPS):
        parts = _sc_agg(P.reshape(T * N, D), idx_p, dst_p, zeros_acc)
        if step < STEPS - 1:
            h, P = _gru_proj_call(parts, h, W_ih, W_hh, b_ih, b_hh, W_e, be3)
        else:
            result, ggnn = _gru_pool_call(parts, h, W_ih, W_hh, b_ih, b_hh,
                                          gids3, wcv, bc)
    return result, ggnn
